# baseline (speedup 1.0000x reference)
import functools

import jax
import jax.numpy as jnp
from jax.experimental import pallas as pl
from jax.experimental.pallas import tpu as pltpu


def _round_up(x, m):
    return ((x + m - 1) // m) * m


def _pool_kernel(batch_ref, x_ref, w_ref, o_ref, acc_ref):
    k = pl.program_id(1)

    @pl.when(k == 0)
    def _init():
        acc_ref[...] = jnp.zeros_like(acc_ref)

    B = acc_ref.shape[0]
    TN = x_ref.shape[0]

    batch = batch_ref[...]
    seg = jax.lax.broadcasted_iota(jnp.int32, (B, TN), 0)
    sel = (seg == batch).astype(jnp.bfloat16)
    xb = x_ref[...].astype(jnp.bfloat16)

    acc_ref[...] += jnp.dot(sel, xb, preferred_element_type=jnp.float32)

    @pl.when(k == pl.num_programs(1) - 1)
    def _head():
        o_ref[0] = jnp.dot(acc_ref[...], w_ref[...],
                           preferred_element_type=jnp.float32)


def _combine_kernel(p_ref, b_ref, o_ref):
    y = p_ref[0] + p_ref[1] + b_ref[...]
    o_ref[...] = jnp.where(y > 0, y, 0.01 * y)


@functools.partial(jax.jit, static_argnames=("num_graphs", "tn"))
def _forward(x, batch, weight, bias, num_graphs, tn=1024):
    N, c_in = x.shape
    c_out = weight.shape[0]
    B = int(num_graphs)
    tn = int(tn)

    n_pad = _round_up(N, 2 * tn)
    nk = n_pad // tn // 2

    x_p = jnp.pad(x.astype(jnp.float32), ((0, n_pad - N), (0, 0)))
    batch_p = jnp.pad(batch.astype(jnp.int32), (0, n_pad - N),
                      constant_values=B).reshape(1, n_pad)
    w_t = weight.T.astype(jnp.float32)
    b2d = bias.reshape(1, c_out).astype(jnp.float32)

    vmem_bytes = (2 * (tn * c_in * 4 + tn * 4)
                  + c_in * c_out * 4 + 2 * B * c_out * 4 + B * c_in * 4
                  + (6 << 20))
    vmem_bytes = min(max(vmem_bytes, 16 << 20), 100 << 20)

    cost = pl.CostEstimate(
        flops=2 * B * n_pad * c_in + 2 * 2 * B * c_in * c_out,
        transcendentals=0,
        bytes_accessed=(n_pad * c_in * 4 + n_pad * 4
                        + c_in * c_out * 4 + 2 * B * c_out * 4),
    )

    partial = pl.pallas_call(
        _pool_kernel,
        out_shape=jax.ShapeDtypeStruct((2, B, c_out), jnp.float32),
        grid_spec=pltpu.PrefetchScalarGridSpec(
            num_scalar_prefetch=0,
            grid=(2, nk),
            in_specs=[
                pl.BlockSpec((1, tn), lambda j, k: (0, j * nk + k)),
                pl.BlockSpec((tn, c_in), lambda j, k: (j * nk + k, 0)),
                pl.BlockSpec((c_in, c_out), lambda j, k: (0, 0)),
            ],
            out_specs=pl.BlockSpec((1, B, c_out), lambda j, k: (j, 0, 0)),
            scratch_shapes=[pltpu.VMEM((B, c_in), jnp.float32)],
        ),
        compiler_params=pltpu.CompilerParams(
            dimension_semantics=("parallel", "arbitrary"),
            vmem_limit_bytes=vmem_bytes,
        ),
        cost_estimate=cost,
    )(batch_p, x_p, w_t)

    return pl.pallas_call(
        _combine_kernel,
        out_shape=jax.ShapeDtypeStruct((B, c_out), jnp.float32),
    )(partial, b2d)


def kernel(x, batch, weight, bias):
    return _forward(x, batch, weight, bias, 128)


# speedup vs baseline: 2.0319x; 2.0319x over previous
import functools

import jax
import jax.numpy as jnp
from jax.experimental import pallas as pl
from jax.experimental.pallas import tpu as pltpu

_T = 4


def _round_up(x, m):
    return ((x + m - 1) // m) * m


def _pool_kernel(*refs):
    b_refs = refs[:_T]
    x_refs = refs[_T:2 * _T]
    w_ref, bias_ref, o_ref, acc_ref = refs[2 * _T:]
    k = pl.program_id(0)

    @pl.when(k == 0)
    def _init():
        acc_ref[...] = jnp.zeros_like(acc_ref)

    B = acc_ref.shape[0]
    TN = x_refs[0].shape[1]

    partial = None
    for i in range(_T):
        batch = b_refs[i][0]
        seg = jax.lax.broadcasted_iota(jnp.int32, (B, TN), 0)
        sel = (seg == batch).astype(jnp.bfloat16)
        xb = x_refs[i][0].astype(jnp.bfloat16)
        d = jnp.dot(sel, xb, preferred_element_type=jnp.float32)
        partial = d if partial is None else partial + d
    acc_ref[...] += partial

    @pl.when(k == pl.num_programs(0) - 1)
    def _head():
        y = jnp.dot(acc_ref[...], w_ref[...],
                    preferred_element_type=jnp.float32) + bias_ref[...]
        o_ref[...] = jnp.where(y > 0, y, 0.01 * y)


@functools.partial(jax.jit, static_argnames=("num_graphs", "tn"))
def _forward(x, batch, weight, bias, num_graphs, tn=1024):
    N, c_in = x.shape
    c_out = weight.shape[0]
    B = int(num_graphs)
    tn = int(tn)

    n_pad = _round_up(N, _T * tn)
    num_tiles = n_pad // tn
    nk = num_tiles // _T

    x_p = jnp.pad(x.astype(jnp.float32), ((0, n_pad - N), (0, 0)))
    x_t = x_p.reshape(num_tiles, tn, c_in)
    batch_p = jnp.pad(batch.astype(jnp.int32), (0, n_pad - N),
                      constant_values=B).reshape(num_tiles, 1, tn)
    w_t = weight.T.astype(jnp.float32)
    b2d = bias.reshape(1, c_out).astype(jnp.float32)

    vmem_bytes = (2 * _T * (tn * c_in * 4 + tn * 4)
                  + c_in * c_out * 4 + B * c_out * 4 + B * c_in * 4
                  + (8 << 20))
    vmem_bytes = min(max(vmem_bytes, 16 << 20), 100 << 20)

    cost = pl.CostEstimate(
        flops=2 * B * n_pad * c_in + 2 * B * c_in * c_out,
        transcendentals=0,
        bytes_accessed=(n_pad * c_in * 4 + n_pad * 4
                        + c_in * c_out * 4 + B * c_out * 4),
    )

    def _b_spec(i):
        return pl.BlockSpec((1, 1, tn), lambda k, i=i: (_T * k + i, 0, 0))

    def _x_spec(i):
        return pl.BlockSpec((1, tn, c_in), lambda k, i=i: (_T * k + i, 0, 0))

    return pl.pallas_call(
        _pool_kernel,
        out_shape=jax.ShapeDtypeStruct((B, c_out), jnp.float32),
        grid_spec=pltpu.PrefetchScalarGridSpec(
            num_scalar_prefetch=0,
            grid=(nk,),
            in_specs=([_b_spec(i) for i in range(_T)]
                      + [_x_spec(i) for i in range(_T)]
                      + [pl.BlockSpec((c_in, c_out), lambda k: (0, 0)),
                         pl.BlockSpec((1, c_out), lambda k: (0, 0))]),
            out_specs=pl.BlockSpec((B, c_out), lambda k: (0, 0)),
            scratch_shapes=[pltpu.VMEM((B, c_in), jnp.float32)],
        ),
        compiler_params=pltpu.CompilerParams(
            dimension_semantics=("arbitrary",),
            vmem_limit_bytes=vmem_bytes,
        ),
        cost_estimate=cost,
    )(*([batch_p] * _T + [x_t] * _T + [w_t, b2d]))


def kernel(x, batch, weight, bias):
    return _forward(x, batch, weight, bias, 128)
